# Initial kernel scaffold; baseline (speedup 1.0000x reference)
import jax
import jax.numpy as jnp
from jax.experimental import pallas as pl


def kernel(x, edge_index, edge_index_2, W1, b1, Wk1_1, bk1_1, Wk1_2, bk1_2, Wk2_1, bk2_1, Wk2_2, bk2_2, Wk3_1, bk3_1, Wk3_2, bk3_2, Wr0, br0, Wr1, br1, Wr2, br2, Wr3, br3, w0, w1, w2, w3):
    raise NotImplementedError("write your pallas kernel here")


# speedup vs baseline: 27.3571x; 27.3571x over previous
import functools

import jax
import jax.numpy as jnp
from jax import lax
from jax.experimental import pallas as pl
from jax.experimental.pallas import tpu as pltpu
import jax.experimental.pallas.tpu_sc as plsc

N = 10000
NP = 10240
E = 320000
D = 128
NC = 2
NS = 16
ROWS = NP // NS
EPT = E // NS
CHB = 128
CH = (EPT + CHB - 1) // CHB
EPAD = CH * CHB
DUMP = N + 8
K = 8

_mesh = plsc.VectorSubcoreMesh(core_axis_name="c", subcore_axis_name="s")
_scp = pltpu.CompilerParams(use_tc_tiling_on_sc=False,
                            needs_layout_passes=False)
_f32 = jnp.float32
_i32 = jnp.int32


def _iota16():
    return lax.broadcasted_iota(_i32, (16,), 0)


def _hist_body(sidx_hbm, dinv_hbm, sidx_v, ones_v, sl_v, acc):
    c = lax.axis_index("c")
    s = lax.axis_index("s")
    sl = pl.ds(s * ROWS, ROWS)
    pltpu.sync_copy(sidx_hbm.at[c, s], sidx_v)
    for i in range(CHB // 16):
        ones_v[pl.ds(16 * i, 16)] = jnp.ones((16,), _f32)
    for i in range(ROWS // 16):
        sl_v[pl.ds(16 * i, 16)] = jnp.zeros((16,), _f32)
    pltpu.sync_copy(sl_v, acc.at[sl])
    plsc.subcore_barrier()

    def body(i, carry):
        pltpu.sync_copy(ones_v, acc.at[sidx_v.at[i]], add=True)
        return carry

    lax.fori_loop(0, CH, body, 0)
    plsc.subcore_barrier()
    pltpu.sync_copy(acc.at[sl], sl_v)
    pltpu.sync_copy(sl_v, dinv_hbm.at[c, sl])


_sc_hist = functools.partial(
    pl.kernel,
    out_type=jax.ShapeDtypeStruct((NC, NP), _f32),
    mesh=_mesh,
    compiler_params=_scp,
    scratch_types=[
        pltpu.VMEM((CH, CHB), _i32),
        pltpu.VMEM((CHB,), _f32),
        pltpu.VMEM((ROWS,), _f32),
        pltpu.VMEM_SHARED((NP,), _f32),
    ],
)(_hist_body)


_BS = 1024
_NBB = NP // _BS


def _tc_body(x_ref, wt_ref, b1_ref, g_ref, wr0_ref, deg_ref,
             x12_ref, z0_ref, dinv_ref):
    r0 = jnp.maximum(
        jnp.dot(x_ref[...], wt_ref[...], preferred_element_type=_f32)
        + b1_ref[...], 0.0)
    z = jnp.dot(r0, g_ref[0], preferred_element_type=_f32)
    deg = deg_ref[0]
    dv = jnp.where(deg > 0.5, lax.rsqrt(jnp.maximum(deg, 1e-12)), 0.0)
    x12_ref[...] = z * dv
    z0_ref[...] = jnp.dot(r0, wr0_ref[...], preferred_element_type=_f32)
    dinv_ref[...] = dv[None]


def _tc_call(xp, w1t, b1r, gstack, wr0w, deg3):
    return pl.pallas_call(
        _tc_body,
        grid=(2, _NBB),
        in_specs=[
            pl.BlockSpec((_BS, D), lambda g, i: (i, 0)),
            pl.BlockSpec((D, D), lambda g, i: (0, 0)),
            pl.BlockSpec((1, D), lambda g, i: (0, 0)),
            pl.BlockSpec((1, D, K), lambda g, i: (g, 0, 0)),
            pl.BlockSpec((D, 1), lambda g, i: (0, 0)),
            pl.BlockSpec((1, _BS, 1), lambda g, i: (g, i, 0)),
        ],
        out_specs=[
            pl.BlockSpec((_BS, K), lambda g, i: (g * _NBB + i, 0)),
            pl.BlockSpec((_BS, 1), lambda g, i: (i, 0)),
            pl.BlockSpec((1, _BS, 1), lambda g, i: (g, i, 0)),
        ],
        out_shape=[
            jax.ShapeDtypeStruct((NC * NP, K), _f32),
            jax.ShapeDtypeStruct((NP, 1), _f32),
            jax.ShapeDtypeStruct((NC, NP, 1), _f32),
        ],
    )(xp, w1t, b1r, gstack, wr0w, deg3)


def _prop_loop(src_hbm, gidx_v, sidx_v, gbuf, acc, sem):
    def body(i, carry):
        pltpu.async_copy(src_hbm.at[gidx_v.at[i]], gbuf, sem).wait()
        pltpu.sync_copy(gbuf, acc.at[sidx_v.at[i]], add=True)
        return carry

    lax.fori_loop(0, CH, body, 0)


def _r1_body(gidx_hbm, sidx_hbm, x12_hbm, zrows_hbm, uraw_hbm,
             gidx_v, sidx_v, gbuf, zb, acc, sem):
    c = lax.axis_index("c")
    s = lax.axis_index("s")
    sl = pl.ds(s * ROWS, ROWS)
    pltpu.sync_copy(gidx_hbm.at[c, s], gidx_v)
    pltpu.sync_copy(sidx_hbm.at[c, s], sidx_v)
    pltpu.sync_copy(zrows_hbm, zb)
    pltpu.sync_copy(zb, acc.at[sl])
    plsc.subcore_barrier()
    _prop_loop(x12_hbm, gidx_v, sidx_v, gbuf, acc, sem)
    plsc.subcore_barrier()
    pltpu.sync_copy(acc.at[sl], zb)
    pltpu.sync_copy(zb, uraw_hbm.at[c, sl])


_sc_r1 = functools.partial(
    pl.kernel,
    out_type=jax.ShapeDtypeStruct((NC, NP, K), _f32),
    mesh=_mesh,
    compiler_params=_scp,
    scratch_types=[
        pltpu.VMEM((CH, CHB), _i32),
        pltpu.VMEM((CH, CHB), _i32),
        pltpu.VMEM((CHB, K), _f32),
        pltpu.VMEM((ROWS, K), _f32),
        pltpu.VMEM_SHARED((NP, K), _f32),
        pltpu.SemaphoreType.DMA,
    ],
)(_r1_body)


def _mk_comb_body(with_out):
    def body(uraw_hbm, dinv_hbm, oin_hbm, consts_hbm, icol_hbm,
             gidx_hbm, sidx_hbm, zrows_hbm,
             xcat_hbm, praw_hbm, oout_hbm,
             u1b, u2b, dv1, dv2, ob, cb, icb, xb,
             gidx_v, sidx_v, gbuf, zb, acc, sem):
        c = lax.axis_index("c")
        s = lax.axis_index("s")
        sl = pl.ds(s * ROWS, ROWS)
        pltpu.sync_copy(uraw_hbm.at[0, sl], u1b)
        pltpu.sync_copy(uraw_hbm.at[1, sl], u2b)
        pltpu.sync_copy(dinv_hbm.at[0, sl], dv1)
        pltpu.sync_copy(dinv_hbm.at[1, sl], dv2)
        pltpu.sync_copy(consts_hbm, cb)
        pltpu.sync_copy(icol_hbm.at[c], icb)
        pltpu.sync_copy(gidx_hbm.at[c, s], gidx_v)
        pltpu.sync_copy(sidx_hbm.at[c, s], sidx_v)
        pltpu.sync_copy(zrows_hbm, zb)
        pltpu.sync_copy(zb, acc.at[sl])
        pltpu.sync_copy(oin_hbm.at[sl], ob)

        iot = _iota16()
        npat = lax.shift_right_logical(iot, 3)
        cpat = lax.bitwise_and(iot, 7)
        czero = lax.axis_index("c") == 0
        icbv = icb[...]
        kv = jnp.where(czero, cb[0], cb[1])

        def comb(i, carry):
            nv = npat + 2 * i
            u1v = plsc.load_gather(u1b, [nv, icbv])
            u2v = plsc.load_gather(u2b, [nv, icbv])
            d1v = plsc.load_gather(dv1, [nv])
            d2v = plsc.load_gather(dv2, [nv])
            dov = jnp.where(czero, d1v, d2v)
            xv = dov * (d1v * u1v + d2v * u2v + kv)
            plsc.store_scatter(xb, [nv, cpat], xv)
            return carry

        lax.fori_loop(0, ROWS // 2, comb, 0)
        pltpu.sync_copy(xb, xcat_hbm.at[pl.ds(c * NP + s * ROWS, ROWS)])

        if with_out:
            @pl.when(c == 1)
            def _():
                z16 = iot * 0

                def oloop(i, carry):
                    nv = iot + 16 * i
                    u10 = plsc.load_gather(u1b, [nv, z16])
                    u20 = plsc.load_gather(u2b, [nv, z16])
                    d1v = dv1[pl.ds(16 * i, 16)]
                    d2v = dv2[pl.ds(16 * i, 16)]
                    ov = ob[pl.ds(16 * i, 16)]
                    res = d1v * u10 + d2v * u20 + cb[2]
                    ob[pl.ds(16 * i, 16)] = ov + cb[3] * res
                    return carry

                lax.fori_loop(0, ROWS // 16, oloop, 0)

        @pl.when(c == 1)
        def _():
            pltpu.sync_copy(ob, oout_hbm.at[sl])
        plsc.subcore_barrier()
        _prop_loop(xcat_hbm, gidx_v, sidx_v, gbuf, acc, sem)
        plsc.subcore_barrier()
        pltpu.sync_copy(acc.at[sl], zb)
        pltpu.sync_copy(zb, praw_hbm.at[c, sl])

    return body


def _mk_comb_call(with_out):
    return functools.partial(
        pl.kernel,
        out_type=(
            jax.ShapeDtypeStruct((NC * NP, K), _f32),
            jax.ShapeDtypeStruct((NC, NP, K), _f32),
            jax.ShapeDtypeStruct((NP,), _f32),
        ),
        mesh=_mesh,
        compiler_params=_scp,
        scratch_types=[
            pltpu.VMEM((ROWS, K), _f32),
            pltpu.VMEM((ROWS, K), _f32),
            pltpu.VMEM((ROWS,), _f32),
            pltpu.VMEM((ROWS,), _f32),
            pltpu.VMEM((ROWS,), _f32),
            pltpu.VMEM((8, 16), _f32),
            pltpu.VMEM((16,), _i32),
            pltpu.VMEM((ROWS, K), _f32),
            pltpu.VMEM((CH, CHB), _i32),
            pltpu.VMEM((CH, CHB), _i32),
            pltpu.VMEM((CHB, K), _f32),
            pltpu.VMEM((ROWS, K), _f32),
            pltpu.VMEM_SHARED((NP, K), _f32),
            pltpu.SemaphoreType.DMA,
        ],
    )(_mk_comb_body(with_out))


_sc_r2 = _mk_comb_call(True)
_sc_r3 = _mk_comb_call(True)


def _fin_body(qraw_hbm, dinv_hbm, oin_hbm, consts_hbm, out_hbm,
              q1b, q2b, dv1, dv2, ob, cb):
    s = lax.axis_index("s")
    c = lax.axis_index("c")
    half = ROWS // 2
    sl = pl.ds(s * ROWS + c * half, half)
    pltpu.sync_copy(qraw_hbm.at[0, sl], q1b)
    pltpu.sync_copy(qraw_hbm.at[1, sl], q2b)
    pltpu.sync_copy(dinv_hbm.at[0, sl], dv1)
    pltpu.sync_copy(dinv_hbm.at[1, sl], dv2)
    pltpu.sync_copy(oin_hbm.at[sl], ob)
    pltpu.sync_copy(consts_hbm, cb)
    iot = _iota16()
    z16 = iot * 0

    def oloop(i, carry):
        nv = iot + 16 * i
        q1v = plsc.load_gather(q1b, [nv, z16])
        q2v = plsc.load_gather(q2b, [nv, z16])
        d1v = dv1[pl.ds(16 * i, 16)]
        d2v = dv2[pl.ds(16 * i, 16)]
        ov = ob[pl.ds(16 * i, 16)]
        ob[pl.ds(16 * i, 16)] = ov + cb[3] * (d1v * q1v + d2v * q2v) + cb[2]
        return carry

    lax.fori_loop(0, half // 16, oloop, 0)
    pltpu.sync_copy(ob, out_hbm.at[sl])


_sc_fin = functools.partial(
    pl.kernel,
    out_type=jax.ShapeDtypeStruct((NP,), _f32),
    mesh=_mesh,
    compiler_params=_scp,
    scratch_types=[
        pltpu.VMEM((ROWS // 2, K), _f32),
        pltpu.VMEM((ROWS // 2, K), _f32),
        pltpu.VMEM((ROWS // 2,), _f32),
        pltpu.VMEM((ROWS // 2,), _f32),
        pltpu.VMEM((ROWS // 2,), _f32),
        pltpu.VMEM((8, 16), _f32),
    ],
)(_fin_body)


def _prep_idx(ei, g):
    r = ei[0].astype(_i32).reshape(NS, EPT)
    cc = ei[1].astype(_i32).reshape(NS, EPT)
    rp = jnp.pad(r, ((0, 0), (0, EPAD - EPT)), constant_values=0)
    cp = jnp.pad(cc, ((0, 0), (0, EPAD - EPT)), constant_values=DUMP)
    return ((rp + g * NP).reshape(NS, CH, CHB),
            cp.reshape(NS, CH, CHB))


def kernel(x, edge_index, edge_index_2, W1, b1, Wk1_1, bk1_1, Wk1_2, bk1_2,
           Wk2_1, bk2_1, Wk2_2, bk2_2, Wk3_1, bk3_1, Wk3_2, bk3_2,
           Wr0, br0, Wr1, br1, Wr2, br2, Wr3, br3, w0, w1, w2, w3):
    f = _f32
    a1, a2 = Wr1[0, :D], Wr1[0, D:]
    u1, u2 = Wr2[0, :D], Wr2[0, D:]
    v1, v2 = Wr3[0, :D], Wr3[0, D:]
    s1 = Wk3_1.T @ v1
    s2 = Wk3_2.T @ v2
    t1 = Wk2_1.T @ u1
    t2 = Wk2_2.T @ u2
    t3 = Wk2_1.T @ s1[:D]
    t4 = Wk2_2.T @ s1[D:]
    t5 = Wk2_1.T @ s2[:D]
    t6 = Wk2_2.T @ s2[D:]
    M1 = jnp.stack([a1, t1[:D], t2[:D], t3[:D], t4[:D], t5[:D], t6[:D]], 1)
    M2 = jnp.stack([a2, t1[D:], t2[D:], t3[D:], t4[D:], t5[D:], t6[D:]], 1)
    G1 = jnp.pad(Wk1_1.T @ M1, ((0, 0), (0, 1)))
    G2 = jnp.pad(Wk1_2.T @ M2, ((0, 0), (0, 1)))
    gstack = jnp.stack([G1, G2]).astype(f)

    kap = [bk1_1 @ t[:D] + bk1_2 @ t[D:] for t in (t1, t2, t3, t4, t5, t6)]
    kap2_s1 = bk2_1 @ s1[:D] + bk2_2 @ s1[D:]
    kap2_s2 = bk2_1 @ s2[:D] + bk2_2 @ s2[D:]
    c1 = bk1_1 @ a1 + bk1_2 @ a2 + br1[0]
    c2 = bk2_1 @ u1 + bk2_2 @ u2 + br2[0]
    c3 = bk3_1 @ v1 + bk3_2 @ v2 + br3[0]
    zs = jnp.float32(0.0)

    def row(vals):
        return jnp.tile(jnp.stack(vals), 2)

    consts2 = jnp.stack([
        row([kap[0], kap[2], kap[4], zs, zs, zs, zs, zs]),
        row([kap[1], kap[3], kap[5], zs, zs, zs, zs, zs]),
        jnp.full((16,), c1, f), jnp.full((16,), w1[0], f),
        jnp.zeros((16,), f), jnp.zeros((16,), f),
        jnp.zeros((16,), f), jnp.zeros((16,), f)]).astype(f)
    consts3 = jnp.stack([
        row([kap2_s1, zs, zs, zs, zs, zs, zs, zs]),
        row([kap2_s2, zs, zs, zs, zs, zs, zs, zs]),
        jnp.full((16,), c2, f), jnp.full((16,), w2[0], f),
        jnp.zeros((16,), f), jnp.zeros((16,), f),
        jnp.zeros((16,), f), jnp.zeros((16,), f)]).astype(f)
    constsf = jnp.stack([
        jnp.zeros((16,), f), jnp.zeros((16,), f),
        jnp.full((16,), w3[0] * c3 + w0[0] * br0[0], f),
        jnp.full((16,), w3[0], f),
        jnp.zeros((16,), f), jnp.zeros((16,), f),
        jnp.zeros((16,), f), jnp.zeros((16,), f)]).astype(f)
    icol2 = jnp.asarray([[1, 3, 5, 7, 7, 7, 7, 7] * 2,
                         [2, 4, 6, 7, 7, 7, 7, 7] * 2], dtype=_i32)
    icol3 = jnp.asarray([[1] + [7] * 7 + [1] + [7] * 7,
                         [2] + [7] * 7 + [2] + [7] * 7], dtype=_i32)

    g1g, g1s = _prep_idx(edge_index, 0)
    g2g, g2s = _prep_idx(edge_index_2, 1)
    gidx = jnp.stack([g1g, g2g])
    sidx = jnp.stack([g1s, g2s])
    zrows = jnp.zeros((ROWS, K), f)

    deg = _sc_hist(sidx)
    xp = jnp.pad(x.astype(f), ((0, NP - N), (0, 0)))
    w1t = W1.T.astype(f)
    b1r = b1.reshape(1, D).astype(f)
    wr0w = (Wr0.T * w0[0]).astype(f)
    x12, z0, dinv3 = _tc_call(xp, w1t, b1r, gstack, wr0w, deg[:, :, None])
    dinv = dinv3[:, :, 0]
    uraw = _sc_r1(gidx, sidx, x12, zrows)
    _, praw, oacc = _sc_r2(uraw, dinv, z0[:, 0], consts2, icol2,
                           gidx, sidx, zrows)
    _, qraw, oacc2 = _sc_r3(praw, dinv, oacc, consts3, icol3,
                            gidx, sidx, zrows)
    outf = _sc_fin(qraw, dinv, oacc2, constsf)
    return outf[:N, None]


# speedup vs baseline: 38.1471x; 1.3944x over previous
import functools

import jax
import jax.numpy as jnp
from jax import lax
from jax.experimental import pallas as pl
from jax.experimental.pallas import tpu as pltpu
import jax.experimental.pallas.tpu_sc as plsc

N = 10000
NP = 10240
E = 320000
D = 128
NC = 2
NS = 16
ROWS = NP // NS
EPT = E // NS
CHB = 128
NBUF = 4
CH = -(-EPT // (CHB * NBUF)) * NBUF
EPAD = CH * CHB
DUMP = N + 8
K = 8

_mesh = plsc.VectorSubcoreMesh(core_axis_name="c", subcore_axis_name="s")
_scp = pltpu.CompilerParams(use_tc_tiling_on_sc=False,
                            needs_layout_passes=False)
_f32 = jnp.float32
_i32 = jnp.int32


def _iota16():
    return lax.broadcasted_iota(_i32, (16,), 0)


def _hist_body(sidx_hbm, dinv_hbm, sidx_v, ones_v, sl_v, acc, hsem):
    c = lax.axis_index("c")
    s = lax.axis_index("s")
    sl = pl.ds(s * ROWS, ROWS)
    pltpu.sync_copy(sidx_hbm.at[c, s], sidx_v)
    for i in range(CHB // 16):
        ones_v[pl.ds(16 * i, 16)] = jnp.ones((16,), _f32)
    for i in range(ROWS // 16):
        sl_v[pl.ds(16 * i, 16)] = jnp.zeros((16,), _f32)
    pltpu.sync_copy(sl_v, acc.at[sl])
    plsc.subcore_barrier()

    def body(j, carry):
        i = NBUF * j
        sd = [pltpu.async_copy(ones_v, acc.at[sidx_v.at[i + k]],
                               hsem, add=True) for k in range(NBUF)]
        for k in range(NBUF):
            sd[k].wait()
        return carry

    lax.fori_loop(0, CH // NBUF, body, 0)
    plsc.subcore_barrier()
    pltpu.sync_copy(acc.at[sl], sl_v)
    pltpu.sync_copy(sl_v, dinv_hbm.at[c, sl])


_sc_hist = functools.partial(
    pl.kernel,
    out_type=jax.ShapeDtypeStruct((NC, NP), _f32),
    mesh=_mesh,
    compiler_params=_scp,
    scratch_types=[
        pltpu.VMEM((CH, CHB), _i32),
        pltpu.VMEM((CHB,), _f32),
        pltpu.VMEM((ROWS,), _f32),
        pltpu.VMEM_SHARED((NP,), _f32),
        pltpu.SemaphoreType.DMA,
    ],
)(_hist_body)


_BS = 1024
_NBB = NP // _BS


def _tc_body(x_ref, wt_ref, b1_ref, g_ref, wr0_ref, deg_ref,
             x12_ref, z0_ref, dinv_ref):
    r0 = jnp.maximum(
        jnp.dot(x_ref[...], wt_ref[...], preferred_element_type=_f32)
        + b1_ref[...], 0.0)
    z = jnp.dot(r0, g_ref[0], preferred_element_type=_f32)
    deg = deg_ref[0]
    dv = jnp.where(deg > 0.5, lax.rsqrt(jnp.maximum(deg, 1e-12)), 0.0)
    x12_ref[...] = z * dv
    z0_ref[...] = jnp.dot(r0, wr0_ref[...], preferred_element_type=_f32)
    dinv_ref[...] = dv[None]


def _tc_call(xp, w1t, b1r, gstack, wr0w, deg3):
    return pl.pallas_call(
        _tc_body,
        grid=(2, _NBB),
        in_specs=[
            pl.BlockSpec((_BS, D), lambda g, i: (i, 0)),
            pl.BlockSpec((D, D), lambda g, i: (0, 0)),
            pl.BlockSpec((1, D), lambda g, i: (0, 0)),
            pl.BlockSpec((1, D, K), lambda g, i: (g, 0, 0)),
            pl.BlockSpec((D, 1), lambda g, i: (0, 0)),
            pl.BlockSpec((1, _BS, 1), lambda g, i: (g, i, 0)),
        ],
        out_specs=[
            pl.BlockSpec((_BS, K), lambda g, i: (g * _NBB + i, 0)),
            pl.BlockSpec((_BS, 1), lambda g, i: (i, 0)),
            pl.BlockSpec((1, _BS, 1), lambda g, i: (g, i, 0)),
        ],
        out_shape=[
            jax.ShapeDtypeStruct((NC * NP, K), _f32),
            jax.ShapeDtypeStruct((NP, 1), _f32),
            jax.ShapeDtypeStruct((NC, NP, 1), _f32),
        ],
    )(xp, w1t, b1r, gstack, wr0w, deg3)


def _prop_loop(src_hbm, gidx_v, sidx_v, gbufs, acc, gsems, ssem):
    def body(j, carry):
        i = NBUF * j
        gd = [pltpu.async_copy(src_hbm.at[gidx_v.at[i + k]], gbufs[k],
                               gsems[k]) for k in range(NBUF)]
        sd = []
        for k in range(NBUF):
            gd[k].wait()
            sd.append(pltpu.async_copy(gbufs[k], acc.at[sidx_v.at[i + k]],
                                       ssem, add=True))
        for k in range(NBUF):
            sd[k].wait()
        return carry

    lax.fori_loop(0, CH // NBUF, body, 0)


def _r1_body(gidx_hbm, sidx_hbm, x12_hbm, zrows_hbm, uraw_hbm,
             gidx_v, sidx_v, gb0, gb1, gb2, gb3, zb, acc,
             gs0, gs1, gs2, gs3, ssem):
    c = lax.axis_index("c")
    s = lax.axis_index("s")
    sl = pl.ds(s * ROWS, ROWS)
    pltpu.sync_copy(gidx_hbm.at[c, s], gidx_v)
    pltpu.sync_copy(sidx_hbm.at[c, s], sidx_v)
    pltpu.sync_copy(zrows_hbm, zb)
    pltpu.sync_copy(zb, acc.at[sl])
    plsc.subcore_barrier()
    _prop_loop(x12_hbm, gidx_v, sidx_v, (gb0, gb1, gb2, gb3), acc,
               (gs0, gs1, gs2, gs3), ssem)
    plsc.subcore_barrier()
    pltpu.sync_copy(acc.at[sl], zb)
    pltpu.sync_copy(zb, uraw_hbm.at[c, sl])


_sc_r1 = functools.partial(
    pl.kernel,
    out_type=jax.ShapeDtypeStruct((NC, NP, K), _f32),
    mesh=_mesh,
    compiler_params=_scp,
    scratch_types=[
        pltpu.VMEM((CH, CHB), _i32),
        pltpu.VMEM((CH, CHB), _i32),
        pltpu.VMEM((CHB, K), _f32),
        pltpu.VMEM((CHB, K), _f32),
        pltpu.VMEM((CHB, K), _f32),
        pltpu.VMEM((CHB, K), _f32),
        pltpu.VMEM((ROWS, K), _f32),
        pltpu.VMEM_SHARED((NP, K), _f32),
        pltpu.SemaphoreType.DMA,
        pltpu.SemaphoreType.DMA,
        pltpu.SemaphoreType.DMA,
        pltpu.SemaphoreType.DMA,
        pltpu.SemaphoreType.DMA,
    ],
)(_r1_body)


def _mk_comb_body(with_out):
    def body(uraw_hbm, dinv_hbm, oin_hbm, consts_hbm, icol_hbm,
             gidx_hbm, sidx_hbm, zrows_hbm,
             xcat_hbm, praw_hbm, oout_hbm,
             u1b, u2b, dv1, dv2, ob, cb, icb, xb,
             gidx_v, sidx_v, gb0, gb1, gb2, gb3, zb, acc,
             gs0, gs1, gs2, gs3, ssem):
        c = lax.axis_index("c")
        s = lax.axis_index("s")
        sl = pl.ds(s * ROWS, ROWS)
        pltpu.sync_copy(uraw_hbm.at[0, sl], u1b)
        pltpu.sync_copy(uraw_hbm.at[1, sl], u2b)
        pltpu.sync_copy(dinv_hbm.at[0, sl], dv1)
        pltpu.sync_copy(dinv_hbm.at[1, sl], dv2)
        pltpu.sync_copy(consts_hbm, cb)
        pltpu.sync_copy(icol_hbm.at[c], icb)
        pltpu.sync_copy(gidx_hbm.at[c, s], gidx_v)
        pltpu.sync_copy(sidx_hbm.at[c, s], sidx_v)
        pltpu.sync_copy(zrows_hbm, zb)
        pltpu.sync_copy(zb, acc.at[sl])
        pltpu.sync_copy(oin_hbm.at[sl], ob)

        iot = _iota16()
        npat = lax.shift_right_logical(iot, 3)
        cpat = lax.bitwise_and(iot, 7)
        czero = lax.axis_index("c") == 0
        icbv = icb[...]
        kv = jnp.where(czero, cb[0], cb[1])

        def comb(i, carry):
            nv = npat + 2 * i
            u1v = plsc.load_gather(u1b, [nv, icbv])
            u2v = plsc.load_gather(u2b, [nv, icbv])
            d1v = plsc.load_gather(dv1, [nv])
            d2v = plsc.load_gather(dv2, [nv])
            dov = jnp.where(czero, d1v, d2v)
            xv = dov * (d1v * u1v + d2v * u2v + kv)
            plsc.store_scatter(xb, [nv, cpat], xv)
            return carry

        lax.fori_loop(0, ROWS // 2, comb, 0)
        pltpu.sync_copy(xb, xcat_hbm.at[pl.ds(c * NP + s * ROWS, ROWS)])

        if with_out:
            @pl.when(c == 1)
            def _():
                z16 = iot * 0

                def oloop(i, carry):
                    nv = iot + 16 * i
                    u10 = plsc.load_gather(u1b, [nv, z16])
                    u20 = plsc.load_gather(u2b, [nv, z16])
                    d1v = dv1[pl.ds(16 * i, 16)]
                    d2v = dv2[pl.ds(16 * i, 16)]
                    ov = ob[pl.ds(16 * i, 16)]
                    res = d1v * u10 + d2v * u20 + cb[2]
                    ob[pl.ds(16 * i, 16)] = ov + cb[3] * res
                    return carry

                lax.fori_loop(0, ROWS // 16, oloop, 0)

        @pl.when(c == 1)
        def _():
            pltpu.sync_copy(ob, oout_hbm.at[sl])
        plsc.subcore_barrier()
        _prop_loop(xcat_hbm, gidx_v, sidx_v, (gb0, gb1, gb2, gb3), acc,
                   (gs0, gs1, gs2, gs3), ssem)
        plsc.subcore_barrier()
        pltpu.sync_copy(acc.at[sl], zb)
        pltpu.sync_copy(zb, praw_hbm.at[c, sl])

    return body


def _mk_comb_call(with_out):
    return functools.partial(
        pl.kernel,
        out_type=(
            jax.ShapeDtypeStruct((NC * NP, K), _f32),
            jax.ShapeDtypeStruct((NC, NP, K), _f32),
            jax.ShapeDtypeStruct((NP,), _f32),
        ),
        mesh=_mesh,
        compiler_params=_scp,
        scratch_types=[
            pltpu.VMEM((ROWS, K), _f32),
            pltpu.VMEM((ROWS, K), _f32),
            pltpu.VMEM((ROWS,), _f32),
            pltpu.VMEM((ROWS,), _f32),
            pltpu.VMEM((ROWS,), _f32),
            pltpu.VMEM((8, 16), _f32),
            pltpu.VMEM((16,), _i32),
            pltpu.VMEM((ROWS, K), _f32),
            pltpu.VMEM((CH, CHB), _i32),
            pltpu.VMEM((CH, CHB), _i32),
            pltpu.VMEM((CHB, K), _f32),
            pltpu.VMEM((CHB, K), _f32),
            pltpu.VMEM((CHB, K), _f32),
            pltpu.VMEM((CHB, K), _f32),
            pltpu.VMEM((ROWS, K), _f32),
            pltpu.VMEM_SHARED((NP, K), _f32),
            pltpu.SemaphoreType.DMA,
            pltpu.SemaphoreType.DMA,
            pltpu.SemaphoreType.DMA,
            pltpu.SemaphoreType.DMA,
            pltpu.SemaphoreType.DMA,
        ],
    )(_mk_comb_body(with_out))


_sc_r2 = _mk_comb_call(True)
_sc_r3 = _mk_comb_call(True)


def _fin_body(qraw_hbm, dinv_hbm, oin_hbm, consts_hbm, out_hbm,
              q1b, q2b, dv1, dv2, ob, cb):
    s = lax.axis_index("s")
    c = lax.axis_index("c")
    half = ROWS // 2
    sl = pl.ds(s * ROWS + c * half, half)
    pltpu.sync_copy(qraw_hbm.at[0, sl], q1b)
    pltpu.sync_copy(qraw_hbm.at[1, sl], q2b)
    pltpu.sync_copy(dinv_hbm.at[0, sl], dv1)
    pltpu.sync_copy(dinv_hbm.at[1, sl], dv2)
    pltpu.sync_copy(oin_hbm.at[sl], ob)
    pltpu.sync_copy(consts_hbm, cb)
    iot = _iota16()
    z16 = iot * 0

    def oloop(i, carry):
        nv = iot + 16 * i
        q1v = plsc.load_gather(q1b, [nv, z16])
        q2v = plsc.load_gather(q2b, [nv, z16])
        d1v = dv1[pl.ds(16 * i, 16)]
        d2v = dv2[pl.ds(16 * i, 16)]
        ov = ob[pl.ds(16 * i, 16)]
        ob[pl.ds(16 * i, 16)] = ov + cb[3] * (d1v * q1v + d2v * q2v) + cb[2]
        return carry

    lax.fori_loop(0, half // 16, oloop, 0)
    pltpu.sync_copy(ob, out_hbm.at[sl])


_sc_fin = functools.partial(
    pl.kernel,
    out_type=jax.ShapeDtypeStruct((NP,), _f32),
    mesh=_mesh,
    compiler_params=_scp,
    scratch_types=[
        pltpu.VMEM((ROWS // 2, K), _f32),
        pltpu.VMEM((ROWS // 2, K), _f32),
        pltpu.VMEM((ROWS // 2,), _f32),
        pltpu.VMEM((ROWS // 2,), _f32),
        pltpu.VMEM((ROWS // 2,), _f32),
        pltpu.VMEM((8, 16), _f32),
    ],
)(_fin_body)


def _prep_idx(ei, g):
    r = ei[0].astype(_i32).reshape(NS, EPT)
    cc = ei[1].astype(_i32).reshape(NS, EPT)
    rp = jnp.pad(r, ((0, 0), (0, EPAD - EPT)), constant_values=0)
    cp = jnp.pad(cc, ((0, 0), (0, EPAD - EPT)), constant_values=DUMP)
    return ((rp + g * NP).reshape(NS, CH, CHB),
            cp.reshape(NS, CH, CHB))


def kernel(x, edge_index, edge_index_2, W1, b1, Wk1_1, bk1_1, Wk1_2, bk1_2,
           Wk2_1, bk2_1, Wk2_2, bk2_2, Wk3_1, bk3_1, Wk3_2, bk3_2,
           Wr0, br0, Wr1, br1, Wr2, br2, Wr3, br3, w0, w1, w2, w3):
    f = _f32
    a1, a2 = Wr1[0, :D], Wr1[0, D:]
    u1, u2 = Wr2[0, :D], Wr2[0, D:]
    v1, v2 = Wr3[0, :D], Wr3[0, D:]
    s1 = Wk3_1.T @ v1
    s2 = Wk3_2.T @ v2
    t1 = Wk2_1.T @ u1
    t2 = Wk2_2.T @ u2
    t3 = Wk2_1.T @ s1[:D]
    t4 = Wk2_2.T @ s1[D:]
    t5 = Wk2_1.T @ s2[:D]
    t6 = Wk2_2.T @ s2[D:]
    M1 = jnp.stack([a1, t1[:D], t2[:D], t3[:D], t4[:D], t5[:D], t6[:D]], 1)
    M2 = jnp.stack([a2, t1[D:], t2[D:], t3[D:], t4[D:], t5[D:], t6[D:]], 1)
    G1 = jnp.pad(Wk1_1.T @ M1, ((0, 0), (0, 1)))
    G2 = jnp.pad(Wk1_2.T @ M2, ((0, 0), (0, 1)))
    gstack = jnp.stack([G1, G2]).astype(f)

    kap = [bk1_1 @ t[:D] + bk1_2 @ t[D:] for t in (t1, t2, t3, t4, t5, t6)]
    kap2_s1 = bk2_1 @ s1[:D] + bk2_2 @ s1[D:]
    kap2_s2 = bk2_1 @ s2[:D] + bk2_2 @ s2[D:]
    c1 = bk1_1 @ a1 + bk1_2 @ a2 + br1[0]
    c2 = bk2_1 @ u1 + bk2_2 @ u2 + br2[0]
    c3 = bk3_1 @ v1 + bk3_2 @ v2 + br3[0]
    zs = jnp.float32(0.0)

    def row(vals):
        return jnp.tile(jnp.stack(vals), 2)

    consts2 = jnp.stack([
        row([kap[0], kap[2], kap[4], zs, zs, zs, zs, zs]),
        row([kap[1], kap[3], kap[5], zs, zs, zs, zs, zs]),
        jnp.full((16,), c1, f), jnp.full((16,), w1[0], f),
        jnp.zeros((16,), f), jnp.zeros((16,), f),
        jnp.zeros((16,), f), jnp.zeros((16,), f)]).astype(f)
    consts3 = jnp.stack([
        row([kap2_s1, zs, zs, zs, zs, zs, zs, zs]),
        row([kap2_s2, zs, zs, zs, zs, zs, zs, zs]),
        jnp.full((16,), c2, f), jnp.full((16,), w2[0], f),
        jnp.zeros((16,), f), jnp.zeros((16,), f),
        jnp.zeros((16,), f), jnp.zeros((16,), f)]).astype(f)
    constsf = jnp.stack([
        jnp.zeros((16,), f), jnp.zeros((16,), f),
        jnp.full((16,), w3[0] * c3 + w0[0] * br0[0], f),
        jnp.full((16,), w3[0], f),
        jnp.zeros((16,), f), jnp.zeros((16,), f),
        jnp.zeros((16,), f), jnp.zeros((16,), f)]).astype(f)
    icol2 = jnp.asarray([[1, 3, 5, 7, 7, 7, 7, 7] * 2,
                         [2, 4, 6, 7, 7, 7, 7, 7] * 2], dtype=_i32)
    icol3 = jnp.asarray([[1] + [7] * 7 + [1] + [7] * 7,
                         [2] + [7] * 7 + [2] + [7] * 7], dtype=_i32)

    g1g, g1s = _prep_idx(edge_index, 0)
    g2g, g2s = _prep_idx(edge_index_2, 1)
    gidx = jnp.stack([g1g, g2g])
    sidx = jnp.stack([g1s, g2s])
    zrows = jnp.zeros((ROWS, K), f)

    deg = _sc_hist(sidx)
    xp = jnp.pad(x.astype(f), ((0, NP - N), (0, 0)))
    w1t = W1.T.astype(f)
    b1r = b1.reshape(1, D).astype(f)
    wr0w = (Wr0.T * w0[0]).astype(f)
    x12, z0, dinv3 = _tc_call(xp, w1t, b1r, gstack, wr0w, deg[:, :, None])
    dinv = dinv3[:, :, 0]
    uraw = _sc_r1(gidx, sidx, x12, zrows)
    _, praw, oacc = _sc_r2(uraw, dinv, z0[:, 0], consts2, icol2,
                           gidx, sidx, zrows)
    _, qraw, oacc2 = _sc_r3(praw, dinv, oacc, consts3, icol3,
                            gidx, sidx, zrows)
    outf = _sc_fin(qraw, dinv, oacc2, constsf)
    return outf[:N, None]
